# Initial kernel scaffold; baseline (speedup 1.0000x reference)
import jax
import jax.numpy as jnp
from jax.experimental import pallas as pl


def kernel(tape, draws, start_pos):
    raise NotImplementedError("write your pallas kernel here")


# speedup vs baseline: 3.8137x; 3.8137x over previous
import jax
import jax.numpy as jnp
from jax.experimental import pallas as pl
from jax.experimental.pallas import tpu as pltpu

_R = 4000


def _body(sp_ref, tape_ref, draws_ref, out_ref):
    i = pl.program_id(0)
    sp = sp_ref[0]
    nb = sp_ref[1]
    dr0 = i * _R - sp
    overlap = (dr0 > -_R) & (dr0 < nb)

    @pl.when(jnp.logical_not(overlap))
    def _copy():
        out_ref[...] = tape_ref[...]

    @pl.when(overlap)
    def _mix():
        off = jnp.clip(dr0 + _R, 0, nb + _R)
        rows = i * _R + jax.lax.broadcasted_iota(jnp.int32, (_R, 64), 0)
        mask = (rows >= sp) & (rows < sp + nb)
        dslice = draws_ref[pl.ds(off, _R), :]
        out_ref[...] = jnp.where(mask, dslice, tape_ref[...])


def kernel(tape, draws, start_pos):
    T, d = tape.shape
    B = draws.shape[0]
    sp = jnp.asarray(start_pos, jnp.int32)
    scal = jnp.stack([sp, jnp.int32(B)])
    draws_pad = jnp.pad(draws, ((_R, _R), (0, 0)))
    grid = (T // _R,)
    out = pl.pallas_call(
        _body,
        grid=grid,
        in_specs=[
            pl.BlockSpec(memory_space=pltpu.SMEM),
            pl.BlockSpec((_R, d), lambda i: (i, 0)),
            pl.BlockSpec((B + 2 * _R, d), lambda i: (0, 0)),
        ],
        out_specs=pl.BlockSpec((_R, d), lambda i: (i, 0)),
        out_shape=jax.ShapeDtypeStruct((T, d), tape.dtype),
    )(scal, tape, draws_pad)
    new_pos = jnp.minimum(sp + B, T)
    return out, new_pos


# speedup vs baseline: 3.8252x; 1.0030x over previous
import jax
import jax.numpy as jnp
from jax.experimental import pallas as pl
from jax.experimental.pallas import tpu as pltpu

_R = 10000


def _body(sp_ref, tape_ref, draws_ref, out_ref):
    i = pl.program_id(0)
    sp = sp_ref[0]
    nb = sp_ref[1]
    dr0 = i * _R - sp
    overlap = (dr0 > -_R) & (dr0 < nb)

    @pl.when(jnp.logical_not(overlap))
    def _copy():
        out_ref[...] = tape_ref[...]

    @pl.when(overlap)
    def _mix():
        off = jnp.clip(dr0 + _R, 0, nb + _R)
        rows = i * _R + jax.lax.broadcasted_iota(jnp.int32, (_R, 64), 0)
        mask = (rows >= sp) & (rows < sp + nb)
        dslice = draws_ref[pl.ds(off, _R), :]
        out_ref[...] = jnp.where(mask, dslice, tape_ref[...])


def kernel(tape, draws, start_pos):
    T, d = tape.shape
    B = draws.shape[0]
    sp = jnp.asarray(start_pos, jnp.int32)
    scal = jnp.stack([sp, jnp.int32(B)])
    draws_pad = jnp.pad(draws, ((_R, _R), (0, 0)))
    grid = (T // _R,)
    out = pl.pallas_call(
        _body,
        grid=grid,
        in_specs=[
            pl.BlockSpec(memory_space=pltpu.SMEM),
            pl.BlockSpec((_R, d), lambda i: (i, 0)),
            pl.BlockSpec((B + 2 * _R, d), lambda i: (0, 0)),
        ],
        out_specs=pl.BlockSpec((_R, d), lambda i: (i, 0)),
        out_shape=jax.ShapeDtypeStruct((T, d), tape.dtype),
    )(scal, tape, draws_pad)
    new_pos = jnp.minimum(sp + B, T)
    return out, new_pos
